# baseline (speedup 1.0000x reference)
import functools

import jax
import jax.numpy as jnp
from jax import lax
from jax.experimental import pallas as pl
from jax.experimental.pallas import tpu as pltpu
from jax.experimental.pallas import tpu_sc as plsc

NW = 32
DEPTH = 2
LANE = 16


SW = 2176
RPI = SW // 8


@functools.lru_cache(maxsize=None)
def _make_compact(n_tab, dim, vocab):
    vpad = (vocab + 127) // 128 * 128
    vtiles = vpad // SW
    frows = vpad // 8
    items_total = n_tab * vtiles
    per_w = -(-items_total // NW)
    per_w += (-per_w) % DEPTH
    out_rows = n_tab * frows + RPI
    mesh = plsc.VectorSubcoreMesh(core_axis_name="c", subcore_axis_name="s")

    @functools.partial(
        pl.kernel,
        out_type=jax.ShapeDtypeStruct((out_rows, 128), jnp.float32),
        mesh=mesh,
        compiler_params=pltpu.CompilerParams(
            needs_layout_passes=False,
            disable_bounds_checks=True,
        ),
        scratch_types=[
            pltpu.VMEM((DEPTH, dim, SW + 1), jnp.float32),
            pltpu.VMEM((RPI, 128), jnp.float32),
        ] + [pltpu.SemaphoreType.DMA] * (DEPTH + 1),
    )
    def compact(tin, out_hbm, tbuf, obuf, *sems):
        rsems, wsem = sems[:DEPTH], sems[DEPTH]
        wid = lax.axis_index("s") * 2 + lax.axis_index("c")
        start = wid * per_w
        dlanes = lax.iota(jnp.int32, LANE)

        def fv(t):
            item = jnp.minimum(start + t, items_total - 1)
            return item // vtiles, item % vtiles

        def slab_copy(t, slot):
            f, vt = fv(t)
            return pltpu.make_async_copy(
                tin.at[f, pl.ds(0, dim),
                       pl.ds(pl.multiple_of(vt * SW, SW), SW)],
                tbuf.at[slot, pl.ds(0, dim), pl.ds(0, SW)], rsems[slot])

        def block_copy(t):
            f, vt = fv(t)
            real = (start + t) < items_total
            r0 = jnp.where(real, f * frows + vt * RPI, n_tab * frows)
            return pltpu.make_async_copy(
                obuf,
                out_hbm.at[pl.ds(pl.multiple_of(r0, 8), RPI)],
                wsem)

        for s in range(DEPTH):
            slab_copy(s, s).start()

        def body(touter, _):
            for sl in range(DEPTH):
                t = touter * DEPTH + sl
                slab_copy(t, sl).wait()

                @pl.when(t >= 1)
                def _w():
                    block_copy(t - 1).wait()

                slot_v = jnp.full((LANE,), sl, jnp.int32)

                @plsc.parallel_loop(0, SW, unroll=8)
                def tbody(p):
                    c = jnp.full((LANE,), p, jnp.int32)
                    vals = plsc.load_gather(tbuf, [slot_v, dlanes, c])
                    obuf[p >> 3, pl.ds((p & 7) * LANE, LANE)] = vals
                block_copy(t).start()

                @pl.when(t + DEPTH < per_w)
                def _f():
                    slab_copy(t + DEPTH, sl).start()
            return 0

        lax.fori_loop(0, per_w // DEPTH, body, 0)
        block_copy(per_w - 1).wait()

    return compact


@functools.lru_cache(maxsize=None)
def _make_gather(n_idx, n_rows_tab, dim, chunk_rows):
    per_w = n_idx // NW
    ci = chunk_rows
    n_chunks = per_w // ci
    n_sub = ci // 128
    mesh = plsc.VectorSubcoreMesh(core_axis_name="c", subcore_axis_name="s")

    @functools.partial(
        pl.kernel,
        out_type=jax.ShapeDtypeStruct((n_idx, dim), jnp.float32),
        mesh=mesh,
        compiler_params=pltpu.CompilerParams(
            use_tc_tiling_on_sc=False,
            disable_bounds_checks=True,
        ),
        scratch_types=[
            pltpu.VMEM((ci,), jnp.int32),
            pltpu.VMEM((ci, dim), jnp.float32),
            pltpu.SemaphoreType.DMA,
        ],
    )
    def gather(table_hbm, idx_hbm, out_hbm, idx_v, rows_v, sem):
        wid = lax.axis_index("s") * 2 + lax.axis_index("c")
        base = wid * per_w

        def chunk_body(c, _):
            off = base + c * ci
            pltpu.sync_copy(idx_hbm.at[pl.ds(off, ci)], idx_v)
            copies = []
            for j in range(n_sub):
                copies.append(pltpu.async_copy(
                    table_hbm.at[idx_v.at[pl.ds(j * 128, 128)]],
                    rows_v.at[pl.ds(j * 128, 128)],
                    sem,
                ))
            for cp in copies:
                cp.wait()
            pltpu.sync_copy(rows_v, out_hbm.at[pl.ds(off, ci)])
            return 0

        lax.fori_loop(0, n_chunks, chunk_body, 0)

    return gather


@functools.lru_cache(maxsize=None)
def _make_mlp(n_rows, d_emb, d_other, h1, h2, block_rows):
    def body(e_ref, xo_ref, w1a_ref, w1b_ref, b1_ref, w2_ref, b2_ref,
             w3_ref, b3_ref, o_ref):
        x = jnp.dot(e_ref[...], w1a_ref[...], preferred_element_type=jnp.float32)
        x += jnp.dot(xo_ref[...], w1b_ref[...], preferred_element_type=jnp.float32)
        x = jnp.maximum(x + b1_ref[...], 0.0)
        x = jnp.dot(x, w2_ref[...], preferred_element_type=jnp.float32)
        x = jnp.maximum(x + b2_ref[...], 0.0)
        o_ref[...] = (jnp.dot(x, w3_ref[...], preferred_element_type=jnp.float32)
                      + b3_ref[...])

    rep = lambda i: (0, 0)
    return pl.pallas_call(
        body,
        grid=(n_rows // block_rows,),
        in_specs=[
            pl.BlockSpec((block_rows, d_emb), lambda i: (i, 0)),
            pl.BlockSpec((block_rows, d_other), lambda i: (i, 0)),
            pl.BlockSpec((d_emb, h1), rep),
            pl.BlockSpec((d_other, h1), rep),
            pl.BlockSpec((1, h1), rep),
            pl.BlockSpec((h1, h2), rep),
            pl.BlockSpec((1, h2), rep),
            pl.BlockSpec((h2, 1), rep),
            pl.BlockSpec((1, 1), rep),
        ],
        out_specs=pl.BlockSpec((block_rows, 1), lambda i: (i, 0)),
        out_shape=jax.ShapeDtypeStruct((n_rows, 1), jnp.float32),
    )


def kernel(x_embed, x_other, tables, W1, b1, W2, b2, W3, b3):
    n_rows, n_fields = x_embed.shape
    n_tab, vocab, dim = tables.shape
    d_emb = n_fields * dim
    d_other = x_other.shape[1]
    h1, h2 = W2.shape
    frows = ((vocab + 127) // 128 * 128 + 7) // 8
    stride = frows * 8

    tin = tables.transpose(0, 2, 1)
    packed = _make_compact(n_tab, dim, vocab)(tin)
    table_rows = packed.reshape(packed.shape[0] * 8, dim)

    idx_flat = (x_embed
                + jnp.arange(n_fields, dtype=jnp.int32) * stride).reshape(-1)
    staged = _make_gather(n_rows * n_fields, table_rows.shape[0], dim, 3328)(
        table_rows, idx_flat)
    embs = staged.reshape(n_rows, d_emb)

    mlp = _make_mlp(n_rows, d_emb, d_other, h1, h2, 1024)
    return mlp(embs, x_other,
               W1[:d_emb], W1[d_emb:], b1.reshape(1, h1),
               W2, b2.reshape(1, h2),
               W3, b3.reshape(1, 1))


# speedup vs baseline: 1.0995x; 1.0995x over previous
import functools

import jax
import jax.numpy as jnp
from jax import lax
from jax.experimental import pallas as pl
from jax.experimental.pallas import tpu as pltpu
from jax.experimental.pallas import tpu_sc as plsc

NW = 32
DEPTH = 2
LANE = 16


SW = 2176
RPI = SW // 8


@functools.lru_cache(maxsize=None)
def _make_compact(n_tab, dim, vocab):
    vpad = (vocab + 127) // 128 * 128
    vtiles = vpad // SW
    frows = vpad // 8
    items_total = n_tab * vtiles
    per_w = -(-items_total // NW)
    per_w += (-per_w) % DEPTH
    out_rows = n_tab * frows + RPI
    mesh = plsc.VectorSubcoreMesh(core_axis_name="c", subcore_axis_name="s")

    @functools.partial(
        pl.kernel,
        out_type=jax.ShapeDtypeStruct((out_rows, 128), jnp.float32),
        mesh=mesh,
        compiler_params=pltpu.CompilerParams(
            needs_layout_passes=False,
            disable_bounds_checks=True,
        ),
        scratch_types=[
            pltpu.VMEM((DEPTH, dim, SW + 1), jnp.float32),
            pltpu.VMEM((RPI, 128), jnp.float32),
        ] + [pltpu.SemaphoreType.DMA] * (DEPTH + 1),
    )
    def compact(tin, out_hbm, tbuf, obuf, *sems):
        rsems, wsem = sems[:DEPTH], sems[DEPTH]
        wid = lax.axis_index("s") * 2 + lax.axis_index("c")
        start = wid * per_w
        dlanes = lax.iota(jnp.int32, LANE)

        def fv(t):
            item = jnp.minimum(start + t, items_total - 1)
            return item // vtiles, item % vtiles

        def slab_copy(t, slot):
            f, vt = fv(t)
            return pltpu.make_async_copy(
                tin.at[f, pl.ds(0, dim),
                       pl.ds(pl.multiple_of(vt * SW, SW), SW)],
                tbuf.at[slot, pl.ds(0, dim), pl.ds(0, SW)], rsems[slot])

        def block_copy(t):
            f, vt = fv(t)
            real = (start + t) < items_total
            r0 = jnp.where(real, f * frows + vt * RPI, n_tab * frows)
            return pltpu.make_async_copy(
                obuf,
                out_hbm.at[pl.ds(pl.multiple_of(r0, 8), RPI)],
                wsem)

        for s in range(DEPTH):
            slab_copy(s, s).start()

        def body(touter, _):
            for sl in range(DEPTH):
                t = touter * DEPTH + sl
                slab_copy(t, sl).wait()

                @pl.when(t >= 1)
                def _w():
                    block_copy(t - 1).wait()

                slot_v = jnp.full((LANE,), sl, jnp.int32)

                @plsc.parallel_loop(0, RPI, unroll=2)
                def tbody(batch):
                    c0 = jnp.full((LANE,), batch * 8, jnp.int32)
                    vals = [plsc.load_gather(tbuf, [slot_v, dlanes, c0 + e])
                            for e in range(8)]
                    for e in range(8):
                        obuf[batch, pl.ds(e * LANE, LANE)] = vals[e]
                block_copy(t).start()

                @pl.when(t + DEPTH < per_w)
                def _f():
                    slab_copy(t + DEPTH, sl).start()
            return 0

        lax.fori_loop(0, per_w // DEPTH, body, 0)
        block_copy(per_w - 1).wait()

    return compact


@functools.lru_cache(maxsize=None)
def _make_gather(n_idx, n_rows_tab, dim, chunk_rows):
    per_w = n_idx // NW
    ci = chunk_rows
    n_chunks = per_w // ci
    n_sub = ci // 128
    mesh = plsc.VectorSubcoreMesh(core_axis_name="c", subcore_axis_name="s")

    @functools.partial(
        pl.kernel,
        out_type=jax.ShapeDtypeStruct((n_idx, dim), jnp.float32),
        mesh=mesh,
        compiler_params=pltpu.CompilerParams(
            use_tc_tiling_on_sc=False,
            disable_bounds_checks=True,
        ),
        scratch_types=[
            pltpu.VMEM((ci,), jnp.int32),
            pltpu.VMEM((ci, dim), jnp.float32),
            pltpu.SemaphoreType.DMA,
        ],
    )
    def gather(table_hbm, idx_hbm, out_hbm, idx_v, rows_v, sem):
        wid = lax.axis_index("s") * 2 + lax.axis_index("c")
        base = wid * per_w

        def chunk_body(c, _):
            off = base + c * ci
            pltpu.sync_copy(idx_hbm.at[pl.ds(off, ci)], idx_v)
            copies = []
            for j in range(n_sub):
                copies.append(pltpu.async_copy(
                    table_hbm.at[idx_v.at[pl.ds(j * 128, 128)]],
                    rows_v.at[pl.ds(j * 128, 128)],
                    sem,
                ))
            for cp in copies:
                cp.wait()
            pltpu.sync_copy(rows_v, out_hbm.at[pl.ds(off, ci)])
            return 0

        lax.fori_loop(0, n_chunks, chunk_body, 0)

    return gather


@functools.lru_cache(maxsize=None)
def _make_mlp(n_rows, d_emb, d_other, h1, h2, block_rows):
    def body(e_ref, xo_ref, w1a_ref, w1b_ref, b1_ref, w2_ref, b2_ref,
             w3_ref, b3_ref, o_ref):
        x = jnp.dot(e_ref[...], w1a_ref[...], preferred_element_type=jnp.float32)
        x += jnp.dot(xo_ref[...], w1b_ref[...], preferred_element_type=jnp.float32)
        x = jnp.maximum(x + b1_ref[...], 0.0)
        x = jnp.dot(x, w2_ref[...], preferred_element_type=jnp.float32)
        x = jnp.maximum(x + b2_ref[...], 0.0)
        o_ref[...] = (jnp.dot(x, w3_ref[...], preferred_element_type=jnp.float32)
                      + b3_ref[...])

    rep = lambda i: (0, 0)
    return pl.pallas_call(
        body,
        grid=(n_rows // block_rows,),
        in_specs=[
            pl.BlockSpec((block_rows, d_emb), lambda i: (i, 0)),
            pl.BlockSpec((block_rows, d_other), lambda i: (i, 0)),
            pl.BlockSpec((d_emb, h1), rep),
            pl.BlockSpec((d_other, h1), rep),
            pl.BlockSpec((1, h1), rep),
            pl.BlockSpec((h1, h2), rep),
            pl.BlockSpec((1, h2), rep),
            pl.BlockSpec((h2, 1), rep),
            pl.BlockSpec((1, 1), rep),
        ],
        out_specs=pl.BlockSpec((block_rows, 1), lambda i: (i, 0)),
        out_shape=jax.ShapeDtypeStruct((n_rows, 1), jnp.float32),
    )


def kernel(x_embed, x_other, tables, W1, b1, W2, b2, W3, b3):
    n_rows, n_fields = x_embed.shape
    n_tab, vocab, dim = tables.shape
    d_emb = n_fields * dim
    d_other = x_other.shape[1]
    h1, h2 = W2.shape
    frows = ((vocab + 127) // 128 * 128 + 7) // 8
    stride = frows * 8

    tin = tables.transpose(0, 2, 1)
    packed = _make_compact(n_tab, dim, vocab)(tin)
    table_rows = packed.reshape(packed.shape[0] * 8, dim)

    idx_flat = (x_embed
                + jnp.arange(n_fields, dtype=jnp.int32) * stride).reshape(-1)
    staged = _make_gather(n_rows * n_fields, table_rows.shape[0], dim, 3328)(
        table_rows, idx_flat)
    embs = staged.reshape(n_rows, d_emb)

    mlp = _make_mlp(n_rows, d_emb, d_other, h1, h2, 1024)
    return mlp(embs, x_other,
               W1[:d_emb], W1[d_emb:], b1.reshape(1, h1),
               W2, b2.reshape(1, h2),
               W3, b3.reshape(1, 1))
